# speedup vs baseline: 1.0040x; 1.0040x over previous
import jax
import jax.numpy as jnp
from jax import lax
from jax.experimental import pallas as pl
from jax.experimental.pallas import tpu as pltpu

_K = 1024
_D = 256
_ROWS = 512


def _vq_argmin_kernel(z_ref, w_ref, out_ref):
    z = z_ref[...]
    w = w_ref[...]
    zsq = jnp.sum(z * z, axis=1, keepdims=True)
    wsq = jnp.sum(w * w, axis=1)
    dot = lax.dot_general(z, w, (((1,), (1,)), ((), ())),
                          preferred_element_type=jnp.float32)
    dists = zsq - 2.0 * dot + wsq[None, :]
    mins = jnp.min(dists, axis=1, keepdims=True)
    kidx = lax.broadcasted_iota(jnp.int32, dists.shape, 1)
    idx = jnp.min(jnp.where(dists == mins, kidx, _K), axis=1)
    out_ref[0, 0, :] = idx


def kernel(z_e_x, W):
    b, hw, d = z_e_x.shape
    n = b * hw
    flat = z_e_x.reshape(n, d)
    num_blocks = n // _ROWS
    out = pl.pallas_call(
        _vq_argmin_kernel,
        grid=(num_blocks,),
        in_specs=[
            pl.BlockSpec((_ROWS, _D), lambda i: (i, 0)),
            pl.BlockSpec((_K, _D), lambda i: (0, 0)),
        ],
        out_specs=pl.BlockSpec((1, 1, _ROWS), lambda i: (i, 0, 0)),
        out_shape=jax.ShapeDtypeStruct((num_blocks, 1, _ROWS), jnp.int32),
    )(flat, W)
    return out.reshape(b, hw)


# speedup vs baseline: 1.7719x; 1.7648x over previous
import jax
import jax.numpy as jnp
from jax import lax
from jax.experimental import pallas as pl
from jax.experimental.pallas import tpu as pltpu

_K = 1024
_D = 256
_ROWS = 1024


def _vq_argmin_kernel(z_ref, w_ref, out_ref, wn_ref, wsq_ref, kidx_ref):
    @pl.when(pl.program_id(0) == 0)
    def _init():
        w = w_ref[...]
        wn_ref[...] = w * (-2.0)
        wsq_ref[...] = jnp.sum(w * w, axis=1)[None, :]
        kidx_ref[...] = lax.broadcasted_iota(
            jnp.int32, (1, _K), 1).astype(jnp.float32)

    z = z_ref[...]
    zsq = jnp.sum(z * z, axis=1, keepdims=True)
    dotn = lax.dot_general(z, wn_ref[...], (((1,), (1,)), ((), ())),
                           preferred_element_type=jnp.float32)
    dists = (zsq + dotn) + wsq_ref[...]
    mins = jnp.min(dists, axis=1, keepdims=True)
    idxf = jnp.min(jnp.where(dists == mins, kidx_ref[...], 2.0 * _K),
                   axis=1, keepdims=True)
    out_ref[...] = idxf.astype(jnp.int32)


def kernel(z_e_x, W):
    b, hw, d = z_e_x.shape
    n = b * hw
    flat = z_e_x.reshape(n, d)
    num_blocks = n // _ROWS
    out = pl.pallas_call(
        _vq_argmin_kernel,
        grid=(num_blocks,),
        in_specs=[
            pl.BlockSpec((_ROWS, _D), lambda i: (i, 0)),
            pl.BlockSpec((_K, _D), lambda i: (0, 0)),
        ],
        out_specs=pl.BlockSpec((_ROWS, 1), lambda i: (i, 0)),
        out_shape=jax.ShapeDtypeStruct((n, 1), jnp.int32),
        scratch_shapes=[
            pltpu.VMEM((_K, _D), jnp.float32),
            pltpu.VMEM((1, _K), jnp.float32),
            pltpu.VMEM((1, _K), jnp.float32),
        ],
    )(flat, W)
    return out.reshape(b, hw)


# speedup vs baseline: 2.3269x; 1.3132x over previous
import jax
import jax.numpy as jnp
from jax import lax
from jax.experimental import pallas as pl
from jax.experimental.pallas import tpu as pltpu

_K = 1024
_D = 256
_ROWS = 6144
_KC = 1024


def _vq_argmin_kernel(z_ref, w_ref, out_ref, wn_ref, wsq_ref, kidx_ref):
    @pl.when(pl.program_id(0) == 0)
    def _init():
        w = w_ref[...]
        wn_ref[...] = w * (-2.0)
        wsq_ref[...] = jnp.sum(w * w, axis=1)[None, :]
        kidx_ref[...] = lax.broadcasted_iota(
            jnp.int32, (1, _K), 1).astype(jnp.float32)

    z = z_ref[...]
    zsq = jnp.sum(z * z, axis=1, keepdims=True)
    kidx = kidx_ref[...]
    val = None
    idx = None
    for c in range(_K // _KC):
        lo = c * _KC
        dotn_c = lax.dot_general(z, wn_ref[lo:lo + _KC, :],
                                 (((1,), (1,)), ((), ())),
                                 preferred_element_type=jnp.float32)
        d_all = (zsq + dotn_c) + wsq_ref[:, lo:lo + _KC]
        for g in range(_KC // 128):
            glo = g * 128
            d_c = d_all[:, glo:glo + 128]
            i_c = jnp.broadcast_to(kidx[:, lo + glo:lo + glo + 128], d_c.shape)
            if val is None:
                val, idx = d_c, i_c
            else:
                idx = jnp.where(d_c < val, i_c, idx)
                val = jnp.minimum(val, d_c)
    m = jnp.min(val, axis=1, keepdims=True)
    idxf = jnp.min(jnp.where(val == m, idx, 2.0 * _K), axis=1, keepdims=True)
    out_ref[...] = idxf.astype(jnp.int32)


def kernel(z_e_x, W):
    b, hw, d = z_e_x.shape
    n = b * hw
    flat = z_e_x.reshape(n, d)
    num_blocks = n // _ROWS
    out = pl.pallas_call(
        _vq_argmin_kernel,
        grid=(num_blocks,),
        in_specs=[
            pl.BlockSpec((_ROWS, _D), lambda i: (i, 0)),
            pl.BlockSpec((_K, _D), lambda i: (0, 0)),
        ],
        out_specs=pl.BlockSpec((_ROWS, 1), lambda i: (i, 0)),
        out_shape=jax.ShapeDtypeStruct((n, 1), jnp.int32),
        compiler_params=pltpu.CompilerParams(
            dimension_semantics=("parallel",)),
        scratch_shapes=[
            pltpu.VMEM((_K, _D), jnp.float32),
            pltpu.VMEM((1, _K), jnp.float32),
            pltpu.VMEM((1, _K), jnp.float32),
        ],
    )(flat, W)
    return out.reshape(b, hw)


# speedup vs baseline: 2.3354x; 1.0036x over previous
import jax
import jax.numpy as jnp
from jax import lax
from jax.experimental import pallas as pl
from jax.experimental.pallas import tpu as pltpu

_K = 1024
_D = 256
_ROWS = 4608
_KC = 1024


def _vq_argmin_kernel(z_ref, w_ref, out_ref, wn_ref, wsq_ref, kidx_ref):
    @pl.when(pl.program_id(0) == 0)
    def _init():
        w = w_ref[...]
        wn_ref[...] = w * (-2.0)
        wsq_ref[...] = jnp.sum(w * w, axis=1)[None, :]
        kidx_ref[...] = lax.broadcasted_iota(
            jnp.int32, (1, _K), 1).astype(jnp.float32)

    z = z_ref[...]
    zsq = jnp.sum(z * z, axis=1, keepdims=True)
    kidx = kidx_ref[...]
    val = None
    idx = None
    for c in range(_K // _KC):
        lo = c * _KC
        dotn_c = lax.dot_general(z, wn_ref[lo:lo + _KC, :],
                                 (((1,), (1,)), ((), ())),
                                 preferred_element_type=jnp.float32)
        d_all = (zsq + dotn_c) + wsq_ref[:, lo:lo + _KC]
        for g in range(_KC // 128):
            glo = g * 128
            d_c = d_all[:, glo:glo + 128]
            i_c = jnp.broadcast_to(kidx[:, lo + glo:lo + glo + 128], d_c.shape)
            if val is None:
                val, idx = d_c, i_c
            else:
                idx = jnp.where(d_c < val, i_c, idx)
                val = jnp.minimum(val, d_c)
    m = jnp.min(val, axis=1, keepdims=True)
    idxf = jnp.min(jnp.where(val == m, idx, 2.0 * _K), axis=1, keepdims=True)
    out_ref[...] = idxf.astype(jnp.int32)


def kernel(z_e_x, W):
    b, hw, d = z_e_x.shape
    n = b * hw
    flat = z_e_x.reshape(n, d)
    num_blocks = n // _ROWS
    out = pl.pallas_call(
        _vq_argmin_kernel,
        grid=(num_blocks,),
        in_specs=[
            pl.BlockSpec((_ROWS, _D), lambda i: (i, 0)),
            pl.BlockSpec((_K, _D), lambda i: (0, 0)),
        ],
        out_specs=pl.BlockSpec((_ROWS, 1), lambda i: (i, 0)),
        out_shape=jax.ShapeDtypeStruct((n, 1), jnp.int32),
        scratch_shapes=[
            pltpu.VMEM((_K, _D), jnp.float32),
            pltpu.VMEM((1, _K), jnp.float32),
            pltpu.VMEM((1, _K), jnp.float32),
        ],
    )(flat, W)
    return out.reshape(b, hw)
